# Initial kernel scaffold; baseline (speedup 1.0000x reference)
import jax
import jax.numpy as jnp
from jax.experimental import pallas as pl


def kernel(x, edge_index, edge_weight, m, f, W1, b1, W2, b2, W3, b3, W4, b4, W5, b5):
    raise NotImplementedError("write your pallas kernel here")


# speedup vs baseline: 11.9784x; 11.9784x over previous
import functools

import jax
import jax.numpy as jnp
from jax import lax
from jax.experimental import pallas as pl
from jax.experimental.pallas import tpu as pltpu
from jax.experimental.pallas import tpu_sc as plsc

_N = 10000
_NPAD = 10240
_E = 320000
_NC = 2
_NS = 16
_NW = _NC * _NS
_CHUNK = 128
_NCHUNK = 79
_EPT = _NCHUNK * _CHUNK
_EPAD = _NW * _EPT
_RPT = _NPAD // _NS

_DOUT = [100, 70, 40, 20, 1]
_P = [112, 80, 48, 32, 16]


def _deg_call(dstp, wp, zeros_row):
    mesh = plsc.VectorSubcoreMesh(core_axis_name="c", subcore_axis_name="s")

    @functools.partial(
        pl.kernel,
        out_type=jax.ShapeDtypeStruct((_NC, _NPAD), jnp.float32),
        mesh=mesh,
        compiler_params=pltpu.CompilerParams(use_tc_tiling_on_sc=False),
        scratch_types=[
            pltpu.VMEM((_NCHUNK, _CHUNK), jnp.int32),
            pltpu.VMEM((_EPT,), jnp.float32),
            pltpu.VMEM_SHARED((_NPAD,), jnp.float32),
        ],
    )
    def deg_k(dst_hbm, w_hbm, z_hbm, out_hbm, dst_t, w_t, acc):
        c = lax.axis_index("c")
        s = lax.axis_index("s")
        wid = c * _NS + s
        pltpu.sync_copy(dst_hbm.at[wid], dst_t)
        pltpu.sync_copy(w_hbm.at[wid], w_t)
        pltpu.sync_copy(z_hbm, acc.at[pl.ds(s * _RPT, _RPT)])
        plsc.subcore_barrier()

        def body(ci, carry):
            pltpu.sync_copy(w_t.at[pl.ds(ci * _CHUNK, _CHUNK)],
                            acc.at[dst_t.at[ci]], add=True)
            return carry

        lax.fori_loop(0, _NCHUNK, body, 0)
        plsc.subcore_barrier()
        pltpu.sync_copy(acc.at[pl.ds(s * _RPT, _RPT)],
                        out_hbm.at[c, pl.ds(s * _RPT, _RPT)])

    return deg_k(dstp, wp, zeros_row)


def _scatter_call(srcp, dstp, wp, xs, zeros_rows, p):
    nv = p // 16
    mesh = plsc.VectorSubcoreMesh(core_axis_name="c", subcore_axis_name="s")

    @functools.partial(
        pl.kernel,
        out_type=jax.ShapeDtypeStruct((_NC, _NPAD, p), jnp.float32),
        mesh=mesh,
        compiler_params=pltpu.CompilerParams(use_tc_tiling_on_sc=False),
        scratch_types=[
            pltpu.VMEM((_NCHUNK, _CHUNK), jnp.int32),
            pltpu.VMEM((_NCHUNK, _CHUNK), jnp.int32),
            pltpu.VMEM((_EPT,), jnp.float32),
            pltpu.VMEM((_CHUNK, p), jnp.float32),
            pltpu.VMEM_SHARED((_NPAD, p), jnp.float32),
        ],
    )
    def scat_k(src_hbm, dst_hbm, w_hbm, xs_hbm, z_hbm, out_hbm,
               src_t, dst_t, w_t, buf, acc):
        c = lax.axis_index("c")
        s = lax.axis_index("s")
        wid = c * _NS + s
        pltpu.sync_copy(src_hbm.at[wid], src_t)
        pltpu.sync_copy(dst_hbm.at[wid], dst_t)
        pltpu.sync_copy(w_hbm.at[wid], w_t)
        pltpu.sync_copy(z_hbm, acc.at[pl.ds(s * _RPT, _RPT)])
        plsc.subcore_barrier()

        def body(ci, carry):
            pltpu.sync_copy(xs_hbm.at[src_t.at[ci]], buf)
            cbase = ci * _CHUNK
            for g in range(_CHUNK // 16):
                wvec = w_t[pl.ds(cbase + g * 16, 16)]
                for j in range(16):
                    e = g * 16 + j
                    jdx = jnp.full((16,), j, dtype=jnp.int32)
                    wb = wvec.at[jdx].get(mode="promise_in_bounds")
                    for v in range(nv):
                        sl = pl.ds(v * 16, 16)
                        buf[e, sl] = buf[e, sl] * wb
            pltpu.sync_copy(buf, acc.at[dst_t.at[ci]], add=True)
            return carry

        lax.fori_loop(0, _NCHUNK, body, 0)
        plsc.subcore_barrier()
        pltpu.sync_copy(acc.at[pl.ds(s * _RPT, _RPT)],
                        out_hbm.at[c, pl.ds(s * _RPT, _RPT)])

    return scat_k(srcp, dstp, wp, xs, zeros_rows)


_BLK = 256


def _mm_first(deg, x, w1p, p1):
    def body(deg_r, x_r, w_r, xs_r, dis_r):
        dis = lax.rsqrt(deg_r[...] + 2.0)
        xw = jnp.dot(x_r[...], w_r[...], preferred_element_type=jnp.float32)
        xs_r[...] = dis * xw
        dis_r[...] = dis

    return pl.pallas_call(
        body,
        grid=(_NPAD // _BLK,),
        in_specs=[
            pl.BlockSpec((_BLK, 1), lambda i: (i, 0)),
            pl.BlockSpec((_BLK, 128), lambda i: (i, 0)),
            pl.BlockSpec((128, p1), lambda i: (0, 0)),
        ],
        out_specs=[
            pl.BlockSpec((_BLK, p1), lambda i: (i, 0)),
            pl.BlockSpec((_BLK, 1), lambda i: (i, 0)),
        ],
        out_shape=[
            jax.ShapeDtypeStruct((_NPAD, p1), jnp.float32),
            jax.ShapeDtypeStruct((_NPAD, 1), jnp.float32),
        ],
    )(deg, x, w1p)


def _mm_mid(acc0, acc1, xs, dis, bp, wpd, pin, pout):
    def body(a0, a1, xs_r, dis_r, b_r, w_r, o_r):
        pre = dis_r[...] * (a0[...] + a1[...] + 2.0 * xs_r[...]) + b_r[...]
        h = jnp.maximum(pre, 0.0)
        o_r[...] = dis_r[...] * jnp.dot(h, w_r[...],
                                        preferred_element_type=jnp.float32)

    return pl.pallas_call(
        body,
        grid=(_NPAD // _BLK,),
        in_specs=[
            pl.BlockSpec((_BLK, pin), lambda i: (i, 0)),
            pl.BlockSpec((_BLK, pin), lambda i: (i, 0)),
            pl.BlockSpec((_BLK, pin), lambda i: (i, 0)),
            pl.BlockSpec((_BLK, 1), lambda i: (i, 0)),
            pl.BlockSpec((1, pin), lambda i: (0, 0)),
            pl.BlockSpec((pin, pout), lambda i: (0, 0)),
        ],
        out_specs=pl.BlockSpec((_BLK, pout), lambda i: (i, 0)),
        out_shape=jax.ShapeDtypeStruct((_NPAD, pout), jnp.float32),
    )(acc0, acc1, xs, dis, bp, wpd)


def _mm_final(acc0, acc1, xs, dis, bp, pin):
    def body(a0, a1, xs_r, dis_r, b_r, o_r):
        o_r[...] = (dis_r[...] * (a0[...] + a1[...] + 2.0 * xs_r[...])
                    + b_r[...])

    return pl.pallas_call(
        body,
        grid=(_NPAD // _BLK,),
        in_specs=[
            pl.BlockSpec((_BLK, pin), lambda i: (i, 0)),
            pl.BlockSpec((_BLK, pin), lambda i: (i, 0)),
            pl.BlockSpec((_BLK, pin), lambda i: (i, 0)),
            pl.BlockSpec((_BLK, 1), lambda i: (i, 0)),
            pl.BlockSpec((1, pin), lambda i: (0, 0)),
        ],
        out_specs=pl.BlockSpec((_BLK, pin), lambda i: (i, 0)),
        out_shape=jax.ShapeDtypeStruct((_NPAD, pin), jnp.float32),
    )(acc0, acc1, xs, dis, bp)


def _pad2(a, rows, cols):
    return jnp.pad(a, ((0, rows - a.shape[0]), (0, cols - a.shape[1])))


def kernel(x, edge_index, edge_weight, m, f,
           W1, b1, W2, b2, W3, b3, W4, b4, W5, b5):
    del m, f
    epad = _EPAD - _E
    srcp = jnp.concatenate(
        [edge_index[0], jnp.zeros((epad,), edge_index.dtype)]
    ).reshape(_NW, _NCHUNK, _CHUNK).astype(jnp.int32)
    dstp = jnp.concatenate(
        [edge_index[1], jnp.zeros((epad,), edge_index.dtype)]
    ).reshape(_NW, _NCHUNK, _CHUNK).astype(jnp.int32)
    wp = jnp.concatenate(
        [edge_weight, jnp.zeros((epad,), edge_weight.dtype)]
    ).reshape(_NW, _EPT)

    xpad = jnp.pad(x, ((0, _NPAD - _N), (0, 0)))

    ws = [W1, W2, W3, W4, W5]
    bs = [b1, b2, b3, b4, b5]
    pin_list = [128] + _P[:-1]
    wpads = [_pad2(ws[i], pin_list[i], _P[i]) for i in range(5)]
    bpads = [jnp.pad(bs[i], (0, _P[i] - bs[i].shape[0])).reshape(1, _P[i])
             for i in range(5)]

    deg2 = _deg_call(dstp, wp, jnp.zeros((_RPT,), jnp.float32))
    deg = (deg2[0] + deg2[1]).reshape(_NPAD, 1)

    xs, dis = _mm_first(deg, xpad, wpads[0], _P[0])
    for l in range(4):
        acc = _scatter_call(srcp, dstp, wp, xs,
                            jnp.zeros((_RPT, _P[l]), jnp.float32), _P[l])
        xs = _mm_mid(acc[0], acc[1], xs, dis, bpads[l], wpads[l + 1],
                     _P[l], _P[l + 1])
    acc = _scatter_call(srcp, dstp, wp, xs,
                        jnp.zeros((_RPT, _P[4]), jnp.float32), _P[4])
    out = _mm_final(acc[0], acc[1], xs, dis, bpads[4], _P[4])
    return out[:_N, :1]
